# baseline (speedup 1.0000x reference)
import dataclasses
import functools

import numpy as np

import jax
import jax.numpy as jnp
from jax import lax
from jax.experimental import pallas as pl
from jax.experimental.pallas import tpu as pltpu
from jax.experimental.pallas import tpu_sc as plsc

N = 10000
E = 320000
D = 128
NUM_REL = 474

NC = 2
NS = 16
NW = NC * NS
EDGES_PER_TILE = E // NW
CHUNK = 50
BLK = 40
NBLK = EDGES_PER_TILE // (CHUNK * BLK)
ACC_GROUP = 40
ACC_NGROUPS = N // ACC_GROUP
DW = D // 2


_REL_COLS = 768
_REL_GRID = (NUM_REL * D) // _REL_COLS


def _relation_body(q_ref, w_ref, b_ref, o_ref):
    y = (jnp.dot(q_ref[...], w_ref[...], preferred_element_type=jnp.float32,
                 precision=lax.Precision.HIGHEST) + b_ref[...])
    o_ref[...] = y.astype(jnp.bfloat16)


def _relation(qpad, w_rel, b_rel):
    out = pl.pallas_call(
        _relation_body,
        grid=(_REL_GRID,),
        in_specs=[
            pl.BlockSpec((16, D), lambda i: (0, 0)),
            pl.BlockSpec((D, _REL_COLS), lambda i: (0, i)),
            pl.BlockSpec((1, _REL_COLS), lambda i: (0, i)),
        ],
        out_specs=pl.BlockSpec((16, _REL_COLS), lambda i: (0, i)),
        out_shape=jax.ShapeDtypeStruct((16, NUM_REL * D), jnp.bfloat16),
    )(qpad, w_rel, b_rel.reshape(1, NUM_REL * D))
    return out[:1].reshape(NUM_REL, D)


_CB = 1000


def _cast_body(x_ref, o_ref):
    o_ref[...] = x_ref[...].astype(jnp.bfloat16)


def _cast(x):
    return pl.pallas_call(
        _cast_body,
        grid=(N // _CB,),
        in_specs=[pl.BlockSpec((_CB, D), lambda i: (i, 0))],
        out_specs=pl.BlockSpec((_CB, D), lambda i: (i, 0)),
        out_shape=jax.ShapeDtypeStruct((N, D), jnp.bfloat16),
    )(x)


def _combine_body(h_ref, p0_ref, p1_ref, x_ref, w_ref, b_ref, o_ref, ob_ref):
    u = p0_ref[...] + p1_ref[...] + x_ref[...]
    a = jnp.concatenate([h_ref[...], u], axis=-1)
    y = (jnp.dot(a, w_ref[...], preferred_element_type=jnp.float32,
                 precision=lax.Precision.HIGHEST) + b_ref[...])
    y = jnp.maximum(y, 0.0) + h_ref[...]
    o_ref[...] = y
    ob_ref[...] = y.astype(jnp.bfloat16)


def _combine(h, p0, p1, x, w_lin, b_lin):
    return pl.pallas_call(
        _combine_body,
        grid=(N // _CB,),
        in_specs=[
            pl.BlockSpec((_CB, D), lambda i: (i, 0)),
            pl.BlockSpec((_CB, D), lambda i: (i, 0)),
            pl.BlockSpec((_CB, D), lambda i: (i, 0)),
            pl.BlockSpec((_CB, D), lambda i: (i, 0)),
            pl.BlockSpec((2 * D, D), lambda i: (0, 0)),
            pl.BlockSpec((1, D), lambda i: (0, 0)),
        ],
        out_specs=[
            pl.BlockSpec((_CB, D), lambda i: (i, 0)),
            pl.BlockSpec((_CB, D), lambda i: (i, 0)),
        ],
        out_shape=[
            jax.ShapeDtypeStruct((N, D), jnp.float32),
            jax.ShapeDtypeStruct((N, D), jnp.bfloat16),
        ],
    )(h, p0, p1, x, w_lin, b_lin.reshape(1, D))


def _final_body(h_ref, p0_ref, p1_ref, x_ref, w_ref, b_ref, q_ref, o_ref):
    u = p0_ref[...] + p1_ref[...] + x_ref[...]
    a = jnp.concatenate([h_ref[...], u], axis=-1)
    y = (jnp.dot(a, w_ref[...], preferred_element_type=jnp.float32,
                 precision=lax.Precision.HIGHEST) + b_ref[...])
    y = jnp.maximum(y, 0.0) + h_ref[...]
    q = jnp.broadcast_to(q_ref[...], y.shape)
    o_ref[...] = jnp.concatenate([y, q], axis=-1)


def _final(h, p0, p1, x, w_lin, b_lin, query):
    return pl.pallas_call(
        _final_body,
        grid=(N // _CB,),
        in_specs=[
            pl.BlockSpec((_CB, D), lambda i: (i, 0)),
            pl.BlockSpec((_CB, D), lambda i: (i, 0)),
            pl.BlockSpec((_CB, D), lambda i: (i, 0)),
            pl.BlockSpec((_CB, D), lambda i: (i, 0)),
            pl.BlockSpec((2 * D, D), lambda i: (0, 0)),
            pl.BlockSpec((1, D), lambda i: (0, 0)),
            pl.BlockSpec((1, D), lambda i: (0, 0)),
        ],
        out_specs=pl.BlockSpec((_CB, 2 * D), lambda i: (i, 0)),
        out_shape=jax.ShapeDtypeStruct((N, 2 * D), jnp.float32),
    )(h, p0, p1, x, w_lin, b_lin.reshape(1, D), query.reshape(1, D))


def _message(h, rel, src2, dst2, et2):
    mesh = plsc.VectorSubcoreMesh(core_axis_name="c", subcore_axis_name="s")
    cp = pltpu.CompilerParams(use_tc_tiling_on_sc=False,
                              needs_layout_passes=False)

    @functools.partial(
        pl.kernel,
        out_type=jax.ShapeDtypeStruct((NC, N, D), jnp.float32),
        mesh=mesh,
        compiler_params=cp,
        scratch_types=[
            pltpu.VMEM((BLK, CHUNK), jnp.int32),
            pltpu.VMEM((BLK, CHUNK), jnp.int32),
            pltpu.VMEM((BLK, CHUNK), jnp.int32),
            pltpu.VMEM((CHUNK, DW), jnp.int32),
            pltpu.VMEM((CHUNK, DW), jnp.int32),
            pltpu.VMEM((CHUNK, DW), jnp.int32),
            pltpu.VMEM((CHUNK, DW), jnp.int32),
            pltpu.VMEM((CHUNK, D), jnp.float32),
            pltpu.VMEM((CHUNK, D), jnp.float32),
            pltpu.VMEM((ACC_GROUP, D), jnp.float32),
            pltpu.VMEM_SHARED((N, D), jnp.float32),
            pltpu.VMEM_SHARED((NUM_REL, DW), jnp.int32),
            pltpu.SemaphoreType.DMA,
            pltpu.SemaphoreType.DMA,
            pltpu.SemaphoreType.DMA,
            pltpu.SemaphoreType.DMA,
            pltpu.SemaphoreType.DMA,
            pltpu.SemaphoreType.DMA,
        ],
    )
    def k(h_hbm, rel_hbm, src_hbm, dst_hbm, et_hbm, out_hbm,
          srcb, dstb, etb, hbufA, hbufB, rbufA, rbufB, mbufA, mbufB,
          zbuf, acc, rel_s,
          semHA, semHB, semRA, semRB, semSA, semSB):
        cid = lax.axis_index("c")
        sid = lax.axis_index("s")
        wid = cid * NS + sid
        hbufs = (hbufA, hbufB)
        rbufs = (rbufA, rbufB)
        mbufs = (mbufA, mbufB)
        semH = (semHA, semHB)
        semR = (semRA, semRB)
        semS = (semSA, semSB)

        zero = jnp.zeros((16,), jnp.float32)

        @pl.loop(0, ACC_GROUP)
        def _zero_zbuf(r):
            for j in range(8):
                zbuf[r, pl.ds(j * 16, 16)] = zero

        for t in range((ACC_NGROUPS + NS - 1) // NS):
            g = t * NS + sid

            @pl.when(g < ACC_NGROUPS)
            def _zero_acc():
                pltpu.sync_copy(zbuf, acc.at[pl.ds(g * ACC_GROUP, ACC_GROUP)])

        @pl.when(sid == 0)
        def _load_rel():
            pltpu.sync_copy(rel_hbm, rel_s)

        plsc.subcore_barrier()

        def issue_gathers(b, row):
            pltpu.async_copy(h_hbm.at[srcb.at[row]], hbufs[b], semH[b])
            pltpu.async_copy(rel_s.at[etb.at[row]], rbufs[b], semR[b])

        def wait_gathers(b):
            pltpu.make_async_copy(h_hbm.at[srcb.at[0]], hbufs[b], semH[b]).wait()
            pltpu.make_async_copy(rel_s.at[etb.at[0]], rbufs[b], semR[b]).wait()

        def issue_scatter(b, row):
            pltpu.async_copy(mbufs[b], acc.at[dstb.at[row]], semS[b], add=True)

        def wait_scatter(b):
            pltpu.make_async_copy(mbufs[b], acc.at[dstb.at[0]], semS[b]).wait()

        def multiply(b):
            hb, rb, mb = hbufs[b], rbufs[b], mbufs[b]

            @pl.loop(0, CHUNK)
            def _mul(r):
                for j in range(4):
                    sl = pl.ds(j * 16, 16)
                    vh = plsc.bitcast(hb[r, sl], jnp.bfloat16)
                    vr = plsc.bitcast(rb[r, sl], jnp.bfloat16)
                    pa, pb = plsc.unpack(vh * vr, format=plsc.PackFormat.INTERLEAVED)
                    mb[r, sl] = pa
                    mb[r, pl.ds(64 + j * 16, 16)] = pb

        tile_row0 = wid * (EDGES_PER_TILE // CHUNK)
        for blk in range(NBLK):
            row0 = tile_row0 + blk * BLK
            pltpu.sync_copy(src_hbm.at[pl.ds(row0, BLK)], srcb)
            pltpu.sync_copy(dst_hbm.at[pl.ds(row0, BLK)], dstb)
            pltpu.sync_copy(et_hbm.at[pl.ds(row0, BLK)], etb)

            issue_gathers(0, 0)
            wait_gathers(0)
            issue_gathers(1, 1)
            multiply(0)
            issue_scatter(0, 0)

            @pl.loop(1, BLK - 1, step=2)
            def _pair(rr):
                wait_gathers(1)
                wait_scatter(0)
                issue_gathers(0, rr + 1)
                multiply(1)
                issue_scatter(1, rr)

                wait_gathers(0)
                wait_scatter(1)
                issue_gathers(1, rr + 2)
                multiply(0)
                issue_scatter(0, rr + 1)

            wait_gathers(1)
            multiply(1)
            issue_scatter(1, BLK - 1)
            wait_scatter(0)
            wait_scatter(1)

        plsc.subcore_barrier()

        for t in range((ACC_NGROUPS + NS - 1) // NS):
            g = t * NS + sid

            @pl.when(g < ACC_NGROUPS)
            def _flush():
                rows = pl.ds(g * ACC_GROUP, ACC_GROUP)
                pltpu.sync_copy(acc.at[rows], out_hbm.at[cid].at[rows])

    return k(h, rel, src2, dst2, et2)


_SIGMA = np.concatenate([np.arange(0, D, 2), np.arange(1, D, 2)])


def _pack(a_bf16):
    n, d = a_bf16.shape
    return jax.lax.bitcast_convert_type(
        a_bf16.reshape(n, d // 2, 2), jnp.int32)


def kernel(x, edge_index, edge_type, query, W_rel_0, b_rel_0, W_lin_0, b_lin_0,
           W_rel_1, b_rel_1, W_lin_1, b_lin_1):
    src2 = edge_index[0].reshape(E // CHUNK, CHUNK)
    dst2 = edge_index[1].reshape(E // CHUNK, CHUNK)
    et2 = edge_type.reshape(E // CHUNK, CHUNK)
    sigma = jnp.asarray(_SIGMA)
    w0p = jnp.concatenate([W_lin_0[:D], W_lin_0[D:][sigma]], axis=0)
    w1p = jnp.concatenate([W_lin_1[:D], W_lin_1[D:][sigma]], axis=0)
    x_perm = x[:, sigma]
    qpad = jnp.zeros((16, D), jnp.float32).at[0].set(query)
    rel0 = _pack(_relation(qpad, W_rel_0, b_rel_0))
    rel1 = _pack(_relation(qpad, W_rel_1, b_rel_1))
    x_pk = _pack(_cast(x))
    parts0 = _message(x_pk, rel0, src2, dst2, et2)
    h1, h1_bf = _combine(x, parts0[0], parts0[1], x_perm, w0p, b_lin_0)
    parts1 = _message(_pack(h1_bf), rel1, src2, dst2, et2)
    return _final(h1, parts1[0], parts1[1], x_perm, w1p, b_lin_1, query)


# speedup vs baseline: 1.5502x; 1.5502x over previous
import functools

import jax
import jax.numpy as jnp
from jax import lax
from jax.experimental import pallas as pl
from jax.experimental.pallas import tpu as pltpu
from jax.experimental.pallas import tpu_sc as plsc

N = 10000
E = 320000
D = 128
NUM_REL = 474

NC = 2
NS = 16
NW = NC * NS
EDGES_PER_TILE = E // NW
CHUNK = 50
BLK = 40
NBLK = EDGES_PER_TILE // (CHUNK * BLK)
ACC_GROUP = 40
ACC_NGROUPS = N // ACC_GROUP


_REL_COLS = 768
_REL_GRID = (NUM_REL * D) // _REL_COLS


def _relation_body(q_ref, w_ref, b_ref, o_ref):
    o_ref[...] = (
        jnp.dot(q_ref[...], w_ref[...], preferred_element_type=jnp.float32,
                precision=lax.Precision.HIGHEST)
        + b_ref[...]
    )


def _relation(qpad, w_rel, b_rel):
    out = pl.pallas_call(
        _relation_body,
        grid=(_REL_GRID,),
        in_specs=[
            pl.BlockSpec((8, D), lambda i: (0, 0)),
            pl.BlockSpec((D, _REL_COLS), lambda i: (0, i)),
            pl.BlockSpec((1, _REL_COLS), lambda i: (0, i)),
        ],
        out_specs=pl.BlockSpec((8, _REL_COLS), lambda i: (0, i)),
        out_shape=jax.ShapeDtypeStruct((8, NUM_REL * D), jnp.float32),
    )(qpad, w_rel, b_rel.reshape(1, NUM_REL * D))
    return out[:1].reshape(NUM_REL, D)


_CB = 1000


def _combine_body(h_ref, p0_ref, p1_ref, x_ref, w_ref, b_ref, o_ref):
    u = p0_ref[...] + p1_ref[...] + x_ref[...]
    a = jnp.concatenate([h_ref[...], u], axis=-1)
    y = (jnp.dot(a, w_ref[...], preferred_element_type=jnp.float32,
                 precision=lax.Precision.HIGHEST) + b_ref[...])
    o_ref[...] = jnp.maximum(y, 0.0) + h_ref[...]


def _combine(h, p0, p1, x, w_lin, b_lin):
    return pl.pallas_call(
        _combine_body,
        grid=(N // _CB,),
        in_specs=[
            pl.BlockSpec((_CB, D), lambda i: (i, 0)),
            pl.BlockSpec((_CB, D), lambda i: (i, 0)),
            pl.BlockSpec((_CB, D), lambda i: (i, 0)),
            pl.BlockSpec((_CB, D), lambda i: (i, 0)),
            pl.BlockSpec((2 * D, D), lambda i: (0, 0)),
            pl.BlockSpec((1, D), lambda i: (0, 0)),
        ],
        out_specs=pl.BlockSpec((_CB, D), lambda i: (i, 0)),
        out_shape=jax.ShapeDtypeStruct((N, D), jnp.float32),
    )(h, p0, p1, x, w_lin, b_lin.reshape(1, D))


def _final_body(h_ref, p0_ref, p1_ref, x_ref, w_ref, b_ref, q_ref, o_ref):
    u = p0_ref[...] + p1_ref[...] + x_ref[...]
    a = jnp.concatenate([h_ref[...], u], axis=-1)
    y = (jnp.dot(a, w_ref[...], preferred_element_type=jnp.float32,
                 precision=lax.Precision.HIGHEST) + b_ref[...])
    y = jnp.maximum(y, 0.0) + h_ref[...]
    q = jnp.broadcast_to(q_ref[...], y.shape)
    o_ref[...] = jnp.concatenate([y, q], axis=-1)


def _final(h, p0, p1, x, w_lin, b_lin, query):
    return pl.pallas_call(
        _final_body,
        grid=(N // _CB,),
        in_specs=[
            pl.BlockSpec((_CB, D), lambda i: (i, 0)),
            pl.BlockSpec((_CB, D), lambda i: (i, 0)),
            pl.BlockSpec((_CB, D), lambda i: (i, 0)),
            pl.BlockSpec((_CB, D), lambda i: (i, 0)),
            pl.BlockSpec((2 * D, D), lambda i: (0, 0)),
            pl.BlockSpec((1, D), lambda i: (0, 0)),
            pl.BlockSpec((1, D), lambda i: (0, 0)),
        ],
        out_specs=pl.BlockSpec((_CB, 2 * D), lambda i: (i, 0)),
        out_shape=jax.ShapeDtypeStruct((N, 2 * D), jnp.float32),
    )(h, p0, p1, x, w_lin, b_lin.reshape(1, D), query.reshape(1, D))


def _message(h, rel, src2, dst2, et2):
    mesh = plsc.VectorSubcoreMesh(core_axis_name="c", subcore_axis_name="s")

    @functools.partial(
        pl.kernel,
        out_type=jax.ShapeDtypeStruct((NC, N, D), jnp.float32),
        mesh=mesh,
        scratch_types=[
            pltpu.VMEM((BLK, CHUNK), jnp.int32),
            pltpu.VMEM((BLK, CHUNK), jnp.int32),
            pltpu.VMEM((BLK, CHUNK), jnp.int32),
            pltpu.VMEM((CHUNK, D), jnp.float32),
            pltpu.VMEM((CHUNK, D), jnp.float32),
            pltpu.VMEM((CHUNK, D), jnp.float32),
            pltpu.VMEM((CHUNK, D), jnp.float32),
            pltpu.VMEM_SHARED((N, D), jnp.float32),
            pltpu.VMEM_SHARED((NUM_REL, D), jnp.float32),
            pltpu.SemaphoreType.DMA,
            pltpu.SemaphoreType.DMA,
            pltpu.SemaphoreType.DMA,
            pltpu.SemaphoreType.DMA,
            pltpu.SemaphoreType.DMA,
            pltpu.SemaphoreType.DMA,
        ],
    )
    def k(h_hbm, rel_hbm, src_hbm, dst_hbm, et_hbm, out_hbm,
          srcb, dstb, etb, hbufA, hbufB, rbufA, rbufB, acc, rel_s,
          semHA, semHB, semRA, semRB, semSA, semSB):
        cid = lax.axis_index("c")
        sid = lax.axis_index("s")
        wid = cid * NS + sid
        hbufs = (hbufA, hbufB)
        rbufs = (rbufA, rbufB)
        semH = (semHA, semHB)
        semR = (semRA, semRB)
        semS = (semSA, semSB)

        zero = jnp.zeros((16,), jnp.float32)

        @pl.loop(0, ACC_GROUP)
        def _zero_hbuf(r):
            for j in range(8):
                hbufA[r, pl.ds(j * 16, 16)] = zero

        for t in range((ACC_NGROUPS + NS - 1) // NS):
            g = t * NS + sid

            @pl.when(g < ACC_NGROUPS)
            def _zero_acc():
                pltpu.sync_copy(hbufA.at[pl.ds(0, ACC_GROUP)],
                                acc.at[pl.ds(g * ACC_GROUP, ACC_GROUP)])

        @pl.when(sid == 0)
        def _load_rel():
            pltpu.sync_copy(rel_hbm, rel_s)

        plsc.subcore_barrier()

        def issue_gathers(b, row):
            pltpu.async_copy(h_hbm.at[srcb.at[row]], hbufs[b], semH[b])
            pltpu.async_copy(rel_s.at[etb.at[row]], rbufs[b], semR[b])

        def wait_gathers(b):
            pltpu.make_async_copy(h_hbm.at[srcb.at[0]], hbufs[b], semH[b]).wait()
            pltpu.make_async_copy(rel_s.at[etb.at[0]], rbufs[b], semR[b]).wait()

        def issue_scatter(b, row):
            pltpu.async_copy(hbufs[b], acc.at[dstb.at[row]], semS[b], add=True)

        def wait_scatter(b):
            pltpu.make_async_copy(hbufs[b], acc.at[dstb.at[0]], semS[b]).wait()

        def multiply(b):
            hb, rb = hbufs[b], rbufs[b]

            @pl.loop(0, CHUNK)
            def _mul(r):
                for j in range(8):
                    sl = pl.ds(j * 16, 16)
                    hb[r, sl] = hb[r, sl] * rb[r, sl]

        tile_row0 = wid * (EDGES_PER_TILE // CHUNK)
        for blk in range(NBLK):
            row0 = tile_row0 + blk * BLK
            pltpu.sync_copy(src_hbm.at[pl.ds(row0, BLK)], srcb)
            pltpu.sync_copy(dst_hbm.at[pl.ds(row0, BLK)], dstb)
            pltpu.sync_copy(et_hbm.at[pl.ds(row0, BLK)], etb)

            issue_gathers(0, 0)
            wait_gathers(0)
            issue_gathers(1, 1)
            multiply(0)
            issue_scatter(0, 0)

            @pl.loop(1, BLK - 1, step=2)
            def _pair(rr):
                wait_gathers(1)
                wait_scatter(0)
                issue_gathers(0, rr + 1)
                multiply(1)
                issue_scatter(1, rr)

                wait_gathers(0)
                wait_scatter(1)
                issue_gathers(1, rr + 2)
                multiply(0)
                issue_scatter(0, rr + 1)

            wait_gathers(1)
            multiply(1)
            issue_scatter(1, BLK - 1)
            wait_scatter(0)
            wait_scatter(1)

        plsc.subcore_barrier()

        for t in range((ACC_NGROUPS + NS - 1) // NS):
            g = t * NS + sid

            @pl.when(g < ACC_NGROUPS)
            def _flush():
                rows = pl.ds(g * ACC_GROUP, ACC_GROUP)
                pltpu.sync_copy(acc.at[rows], out_hbm.at[cid].at[rows])

    return k(h, rel, src2, dst2, et2)


def kernel(x, edge_index, edge_type, query, W_rel_0, b_rel_0, W_lin_0, b_lin_0,
           W_rel_1, b_rel_1, W_lin_1, b_lin_1):
    src2 = edge_index[0].reshape(E // CHUNK, CHUNK)
    dst2 = edge_index[1].reshape(E // CHUNK, CHUNK)
    et2 = edge_type.reshape(E // CHUNK, CHUNK)
    qpad = jnp.zeros((8, D), jnp.float32).at[0].set(query)
    rel0 = _relation(qpad, W_rel_0, b_rel_0)
    rel1 = _relation(qpad, W_rel_1, b_rel_1)
    parts0 = _message(x, rel0, src2, dst2, et2)
    h1 = _combine(x, parts0[0], parts0[1], x, W_lin_0, b_lin_0)
    parts1 = _message(h1, rel1, src2, dst2, et2)
    return _final(h1, parts1[0], parts1[1], x, W_lin_1, b_lin_1, query)
